# Initial kernel scaffold; baseline (speedup 1.0000x reference)
import jax
import jax.numpy as jnp
from jax.experimental import pallas as pl


def kernel(x, non_pad_mask, tables):
    raise NotImplementedError("write your pallas kernel here")


# speedup vs baseline: 10.1099x; 10.1099x over previous
import dataclasses
import functools

import jax
import jax.numpy as jnp
from jax import lax
from jax.experimental import pallas as pl
from jax.experimental.pallas import tpu as pltpu
from jax.experimental.pallas import tpu_sc as plsc

B, L, K, V, D = 1024, 20, 26, 1001, 128
N = B * L * K
NC, NS = 2, 16
NW = NC * NS
PER_W = N // NW
C = 256
NG = C // 128
NCHUNK = PER_W // C

_mesh = plsc.VectorSubcoreMesh(core_axis_name="c", subcore_axis_name="s")

_cp = pltpu.CompilerParams()
if "needs_layout_passes" in pltpu.CompilerParams.__dataclass_fields__:
    _cp = dataclasses.replace(_cp, needs_layout_passes=False)


@functools.partial(
    pl.kernel,
    out_type=jax.ShapeDtypeStruct((N, D), jnp.float32),
    mesh=_mesh,
    compiler_params=_cp,
    scratch_types=[
        pltpu.VMEM((C,), jnp.int32),
        pltpu.VMEM((NG, 128), jnp.int32),
        pltpu.VMEM((C,), jnp.float32),
        pltpu.VMEM((C, D), jnp.float32),
        pltpu.SemaphoreType.DMA,
    ],
)
def _sc_lookup(x_hbm, m_hbm, t_hbm, out_hbm, idx_v, gidx_v, mask_v, rows_v, sem):
    wid = lax.axis_index("s") * NC + lax.axis_index("c")
    w_base = wid * PER_W
    lane = lax.iota(jnp.int32, 16)

    @pl.loop(0, NCHUNK)
    def _chunk(c):
        base = w_base + c * C
        pltpu.sync_copy(x_hbm.at[pl.ds(base, C)], idx_v)
        pltpu.sync_copy(m_hbm.at[pl.ds(base, C)], mask_v)

        for i in range(C // 16):
            pos = lane + (base + i * 16)
            fld = lax.rem(pos, K)
            g = idx_v[pl.ds(i * 16, 16)] + fld * V
            gidx_v[i // 8, pl.ds((i % 8) * 16, 16)] = g

        cps = [
            pltpu.async_copy(
                t_hbm.at[gidx_v.at[j]],
                rows_v.at[pl.ds(j * 128, 128)],
                sem,
            )
            for j in range(NG)
        ]
        for cp in cps:
            cp.wait()

        @pl.loop(0, C, step=16)
        def _grp(g0):
            mv = mask_v[pl.ds(g0, 16)]
            for r in range(16):
                mval = jnp.sum(jnp.where(lane == r, mv, 0.0))
                scale = jnp.full((16,), mval, jnp.float32)
                for j in range(D // 16):
                    sl = pl.ds(j * 16, 16)
                    rows_v[g0 + r, sl] = rows_v[g0 + r, sl] * scale

        pltpu.sync_copy(rows_v, out_hbm.at[pl.ds(base, C)])


def kernel(x, non_pad_mask, tables):
    x_flat = x.reshape(N).astype(jnp.int32)
    m_flat = non_pad_mask.reshape(N)
    t_flat = tables.reshape(K * V, D)
    out = _sc_lookup(x_flat, m_flat, t_flat)
    return out.reshape(B, L, K, D)


# speedup vs baseline: 11.6202x; 1.1494x over previous
import dataclasses
import functools

import jax
import jax.numpy as jnp
from jax import lax
from jax.experimental import pallas as pl
from jax.experimental.pallas import tpu as pltpu
from jax.experimental.pallas import tpu_sc as plsc

B, L, K, V, D = 1024, 20, 26, 1001, 128
N = B * L * K
NC, NS = 2, 16
NW = NC * NS
PER_W = N // NW
C = 128
NCHUNK = PER_W // C

_mesh = plsc.VectorSubcoreMesh(core_axis_name="c", subcore_axis_name="s")

_cp = pltpu.CompilerParams()
if "needs_layout_passes" in pltpu.CompilerParams.__dataclass_fields__:
    _cp = dataclasses.replace(_cp, needs_layout_passes=False)


@functools.partial(
    pl.kernel,
    out_type=jax.ShapeDtypeStruct((N, D), jnp.float32),
    mesh=_mesh,
    compiler_params=_cp,
    scratch_types=[
        pltpu.VMEM((C,), jnp.int32),
        pltpu.VMEM((C,), jnp.int32),
        pltpu.VMEM((C,), jnp.int32),
        pltpu.VMEM((C,), jnp.int32),
        pltpu.VMEM((C,), jnp.float32),
        pltpu.VMEM((C,), jnp.float32),
        pltpu.VMEM((C, D), jnp.float32),
        pltpu.VMEM((C, D), jnp.float32),
        pltpu.SemaphoreType.DMA,
        pltpu.SemaphoreType.DMA,
        pltpu.SemaphoreType.DMA,
    ],
)
def _sc_lookup(x_hbm, m_hbm, t_hbm, out_hbm, idx0, idx1, gidx0, gidx1,
               mask0, mask1, rows0, rows1, gsem0, gsem1, osem):
    wid = lax.axis_index("s") * NC + lax.axis_index("c")
    w_base = wid * PER_W
    lane = lax.iota(jnp.int32, 16)
    idx = (idx0, idx1)
    gidx = (gidx0, gidx1)
    mask = (mask0, mask1)
    rows = (rows0, rows1)
    gsem = (gsem0, gsem1)

    def load_and_index(c, b):
        base = w_base + c * C
        pltpu.sync_copy(x_hbm.at[pl.ds(base, C)], idx[b])
        pltpu.sync_copy(m_hbm.at[pl.ds(base, C)], mask[b])
        for i in range(C // 16):
            pos = lane + (base + i * 16)
            fld = lax.rem(pos, K)
            sl = pl.ds(i * 16, 16)
            gidx[b][sl] = idx[b][sl] + fld * V

    def multiply(b):
        @pl.loop(0, C, step=16)
        def _grp(g0):
            mv = mask[b][pl.ds(g0, 16)]
            for r in range(16):
                mval = jnp.sum(jnp.where(lane == r, mv, 0.0))
                scale = jnp.full((16,), mval, jnp.float32)
                for j in range(D // 16):
                    sl = pl.ds(j * 16, 16)
                    rows[b][g0 + r, sl] = rows[b][g0 + r, sl] * scale

    load_and_index(0, 0)
    pltpu.async_copy(t_hbm.at[gidx[0]], rows[0], gsem[0])
    load_and_index(1, 1)

    @pl.loop(0, NCHUNK, step=2)
    def _pair(c):
        for b in (0, 1):
            cc = c + b
            o = 1 - b

            @pl.when(cc > 0)
            def _():
                pltpu.make_async_copy(
                    rows[o], out_hbm.at[pl.ds(w_base, C)], osem
                ).wait()

            @pl.when(cc + 1 < NCHUNK)
            def _():
                pltpu.async_copy(t_hbm.at[gidx[o]], rows[o], gsem[o])

            pltpu.make_async_copy(t_hbm.at[gidx[b]], rows[b], gsem[b]).wait()
            multiply(b)
            base = w_base + cc * C
            pltpu.async_copy(rows[b], out_hbm.at[pl.ds(base, C)], osem)

            @pl.when(cc + 2 < NCHUNK)
            def _():
                load_and_index(cc + 2, b)

    pltpu.make_async_copy(rows[1], out_hbm.at[pl.ds(w_base, C)], osem).wait()


def kernel(x, non_pad_mask, tables):
    x_flat = x.reshape(N).astype(jnp.int32)
    m_flat = non_pad_mask.reshape(N)
    t_flat = tables.reshape(K * V, D)
    out = _sc_lookup(x_flat, m_flat, t_flat)
    return out.reshape(B, L, K, D)


# speedup vs baseline: 12.2980x; 1.0583x over previous
import dataclasses
import functools

import jax
import jax.numpy as jnp
from jax import lax
from jax.experimental import pallas as pl
from jax.experimental.pallas import tpu as pltpu
from jax.experimental.pallas import tpu_sc as plsc

B, L, K, V, D = 1024, 20, 26, 1001, 128
N = B * L * K
NC, NS = 2, 16
NW = NC * NS
PER_W = N // NW
C = 128
NCHUNK = PER_W // C

_mesh = plsc.VectorSubcoreMesh(core_axis_name="c", subcore_axis_name="s")

_cp = pltpu.CompilerParams()
if "needs_layout_passes" in pltpu.CompilerParams.__dataclass_fields__:
    _cp = dataclasses.replace(_cp, needs_layout_passes=False)


@functools.partial(
    pl.kernel,
    out_type=jax.ShapeDtypeStruct((N, D), jnp.float32),
    mesh=_mesh,
    compiler_params=_cp,
    scratch_types=[
        pltpu.VMEM((C,), jnp.int32),
        pltpu.VMEM((C,), jnp.int32),
        pltpu.VMEM((C,), jnp.int32),
        pltpu.VMEM((C,), jnp.int32),
        pltpu.VMEM((C,), jnp.float32),
        pltpu.VMEM((C,), jnp.float32),
        pltpu.VMEM((C, D), jnp.float32),
        pltpu.VMEM((C, D), jnp.float32),
        pltpu.SemaphoreType.DMA,
        pltpu.SemaphoreType.DMA,
        pltpu.SemaphoreType.DMA,
        pltpu.SemaphoreType.DMA,
        pltpu.SemaphoreType.DMA,
        pltpu.SemaphoreType.DMA,
        pltpu.SemaphoreType.DMA,
    ],
)
def _sc_lookup(x_hbm, m_hbm, t_hbm, out_hbm, idx0, idx1, gidx0, gidx1,
               mask0, mask1, rows0, rows1, gsem0, gsem1, lsem0, lsem1,
               msem0, msem1, osem):
    wid = lax.axis_index("s") * NC + lax.axis_index("c")
    w_base = wid * PER_W
    lane = lax.iota(jnp.int32, 16)
    idx = (idx0, idx1)
    gidx = (gidx0, gidx1)
    mask = (mask0, mask1)
    rows = (rows0, rows1)
    gsem = (gsem0, gsem1)
    lsem = (lsem0, lsem1)
    msem = (msem0, msem1)

    def start_idx_load(c, b):
        pltpu.async_copy(x_hbm.at[pl.ds(w_base + c * C, C)], idx[b], lsem[b])

    def start_mask_load(c, b):
        pltpu.async_copy(m_hbm.at[pl.ds(w_base + c * C, C)], mask[b], msem[b])

    def finish_idx(c, b):
        base = w_base + c * C
        pltpu.make_async_copy(
            x_hbm.at[pl.ds(base, C)], idx[b], lsem[b]).wait()
        for i in range(C // 16):
            pos = lane + (base + i * 16)
            fld = lax.rem(pos, K)
            sl = pl.ds(i * 16, 16)
            gidx[b][sl] = idx[b][sl] + fld * V

    def wait_mask(b):
        pltpu.make_async_copy(
            m_hbm.at[pl.ds(w_base, C)], mask[b], msem[b]).wait()

    def multiply(b):
        @pl.loop(0, C, step=16)
        def _grp(g0):
            mv = mask[b][pl.ds(g0, 16)]
            for r in range(16):
                scale = mv.at[jnp.full((16,), r, jnp.int32)].get(
                    mode="promise_in_bounds")
                for j in range(D // 16):
                    sl = pl.ds(j * 16, 16)
                    rows[b][g0 + r, sl] = rows[b][g0 + r, sl] * scale

    start_idx_load(0, 0)
    start_mask_load(0, 0)
    start_idx_load(1, 1)
    start_mask_load(1, 1)
    finish_idx(0, 0)
    pltpu.async_copy(t_hbm.at[gidx[0]], rows[0], gsem[0])

    @pl.loop(0, NCHUNK, step=2)
    def _pair(c):
        for b in (0, 1):
            cc = c + b
            o = 1 - b

            @pl.when(cc > 0)
            def _():
                pltpu.make_async_copy(
                    rows[o], out_hbm.at[pl.ds(w_base, C)], osem
                ).wait()

            @pl.when(cc + 1 < NCHUNK)
            def _():
                finish_idx(cc + 1, o)
                pltpu.async_copy(t_hbm.at[gidx[o]], rows[o], gsem[o])

            pltpu.make_async_copy(t_hbm.at[gidx[b]], rows[b], gsem[b]).wait()

            @pl.when(cc + 2 < NCHUNK)
            def _():
                start_idx_load(cc + 2, b)

            wait_mask(b)
            multiply(b)
            base = w_base + cc * C
            pltpu.async_copy(rows[b], out_hbm.at[pl.ds(base, C)], osem)

            @pl.when(cc + 2 < NCHUNK)
            def _():
                start_mask_load(cc + 2, b)

    pltpu.make_async_copy(rows[1], out_hbm.at[pl.ds(w_base, C)], osem).wait()


def kernel(x, non_pad_mask, tables):
    x_flat = x.reshape(N).astype(jnp.int32)
    m_flat = non_pad_mask.reshape(N)
    t_flat = tables.reshape(K * V, D)
    out = _sc_lookup(x_flat, m_flat, t_flat)
    return out.reshape(B, L, K, D)
